# baseline (speedup 1.0000x reference)
import jax
import jax.numpy as jnp
from jax.experimental import pallas as pl
from jax.experimental.pallas import tpu as pltpu

N = 10000
BR = 400


def _pass1(adj_ref, x_ref, w1_ref, b1_ref, h_ref, s_ref):
    @pl.when(pl.program_id(0) == 0)
    def _():
        s = jnp.dot(
            x_ref[...].astype(jnp.bfloat16),
            w1_ref[...].astype(jnp.bfloat16),
            preferred_element_type=jnp.float32,
        )
        s_ref[...] = s.astype(jnp.bfloat16)

    a = adj_ref[...].astype(jnp.bfloat16)
    h = jnp.dot(a, s_ref[...], preferred_element_type=jnp.float32)
    h_ref[...] = jnp.maximum(h + b1_ref[...], 0.0)


def _pass2(adj_ref, h_ref, w2_ref, b2_ref, out_ref, v_ref):
    @pl.when(pl.program_id(0) == 0)
    def _():
        v = jnp.dot(
            h_ref[...].astype(jnp.bfloat16),
            w2_ref[...].astype(jnp.bfloat16),
            preferred_element_type=jnp.float32,
        )
        v_ref[...] = v.astype(jnp.bfloat16)

    a = adj_ref[...].astype(jnp.bfloat16)
    out = jnp.dot(a, v_ref[...], preferred_element_type=jnp.float32)
    out_ref[...] = out + b2_ref[...]


@jax.jit
def kernel(x, adj, W1, b1, W2, b2):
    f_in = x.shape[1]
    hid = W1.shape[1]
    ncls = W2.shape[1]
    grid = (N // BR,)

    b1r = b1.reshape(1, hid)
    b2r = b2.reshape(1, ncls)

    h = pl.pallas_call(
        _pass1,
        grid=grid,
        in_specs=[
            pl.BlockSpec((BR, N), lambda i: (i, 0)),
            pl.BlockSpec((N, f_in), lambda i: (0, 0)),
            pl.BlockSpec((f_in, hid), lambda i: (0, 0)),
            pl.BlockSpec((1, hid), lambda i: (0, 0)),
        ],
        out_specs=pl.BlockSpec((BR, hid), lambda i: (i, 0)),
        out_shape=jax.ShapeDtypeStruct((N, hid), jnp.float32),
        scratch_shapes=[pltpu.VMEM((N, hid), jnp.bfloat16)],
    )(adj, x, W1, b1r)

    out = pl.pallas_call(
        _pass2,
        grid=grid,
        in_specs=[
            pl.BlockSpec((BR, N), lambda i: (i, 0)),
            pl.BlockSpec((N, hid), lambda i: (0, 0)),
            pl.BlockSpec((hid, ncls), lambda i: (0, 0)),
            pl.BlockSpec((1, ncls), lambda i: (0, 0)),
        ],
        out_specs=pl.BlockSpec((BR, ncls), lambda i: (i, 0)),
        out_shape=jax.ShapeDtypeStruct((N, ncls), jnp.float32),
        scratch_shapes=[pltpu.VMEM((N, ncls), jnp.bfloat16)],
    )(adj, h, W2, b2r)
    return out
